# baseline (speedup 1.0000x reference)
import functools

import jax
import jax.numpy as jnp
from jax import lax
from jax.experimental import pallas as pl
from jax.experimental.pallas import tpu as pltpu
from jax.experimental.pallas import tpu_sc as plsc

NC = 2
NS = 16
NW = NC * NS
L = 16
PH = 2


def kernel(users, movies, user_emb, movie_emb, W, b):
    B = users.shape[0]
    D = user_emb.shape[1]
    KD = D // L
    bpw = B // NW
    ck = bpw // PH
    NGP = ck // L

    wb = jnp.concatenate([W.reshape(D), jnp.broadcast_to(b, (L,))])

    mesh = plsc.VectorSubcoreMesh(core_axis_name="c", subcore_axis_name="s")

    @functools.partial(
        pl.kernel,
        mesh=mesh,
        compiler_params=pltpu.CompilerParams(
            needs_layout_passes=False, use_tc_tiling_on_sc=True),
        out_type=jax.ShapeDtypeStruct((B,), jnp.float32),
        scratch_types=[
            pltpu.VMEM((bpw,), jnp.int32),
            pltpu.VMEM((bpw,), jnp.int32),
            pltpu.VMEM((ck, 64), jnp.float32),
            pltpu.VMEM((ck, 64), jnp.float32),
            pltpu.VMEM((bpw,), jnp.float32),
            pltpu.VMEM((D + L,), jnp.float32),
            pltpu.SemaphoreType.DMA,
            pltpu.SemaphoreType.DMA,
        ],
    )
    def gmf(users_h, movies_h, uemb_h, memb_h, wb_h, out_h,
            utid, mtid, ubuf, mbuf, outv, wv, usem, msem):
        wid = lax.axis_index("s") * NC + lax.axis_index("c")
        base = wid * bpw

        pltpu.sync_copy(users_h.at[pl.ds(base, bpw)], utid)
        pltpu.sync_copy(movies_h.at[pl.ds(base, bpw)], mtid)
        pltpu.sync_copy(wb_h, wv)

        wvec = [wv[pl.ds(k * L, L)] for k in range(KD)]
        bvec = wv[pl.ds(D, L)]
        lane = lax.iota(jnp.int32, L)

        def run_phase(p, _):
            off = p * ck

            def fire_body(g, _):
                uv = utid[pl.ds(off + g * L, L)]
                mv = mtid[pl.ds(off + g * L, L)]
                for r in range(L):
                    row = g * L + r
                    pltpu.make_async_copy(
                        uemb_h.at[uv[r]], ubuf.at[row], usem).start()
                    pltpu.make_async_copy(
                        memb_h.at[mv[r]], mbuf.at[row], msem).start()
                return 0

            lax.fori_loop(0, NGP, fire_body, 0)

            def drain_body(g, _):
                for r in range(L):
                    row = g * L + r
                    pltpu.make_async_copy(
                        uemb_h.at[0], ubuf.at[row], usem).wait()
                    pltpu.make_async_copy(
                        memb_h.at[0], mbuf.at[row], msem).wait()
                return 0

            lax.fori_loop(0, NGP, drain_body, 0)

            def group_body(g, _):
                rows = g * L + lane
                acc = bvec
                for d in range(D):
                    dsplat = jnp.full((L,), d, jnp.int32)
                    u_d = plsc.load_gather(ubuf, [rows, dsplat])
                    m_d = plsc.load_gather(mbuf, [rows, dsplat])
                    acc = acc + (u_d * m_d) * wvec[d // L][d % L]
                outv[pl.ds(off + g * L, L)] = acc
                return 0

            lax.fori_loop(0, NGP, group_body, 0)
            return 0

        lax.fori_loop(0, PH, run_phase, 0)

        pltpu.sync_copy(outv, out_h.at[pl.ds(base, bpw)])

    out = gmf(users, movies, user_emb, movie_emb, wb)
    return out.reshape(B, 1)
